# baseline (speedup 1.0000x reference)
import functools

import jax
import jax.numpy as jnp
from jax import lax
from jax.experimental import pallas as pl
from jax.experimental.pallas import tpu as pltpu
from jax.experimental.pallas import tpu_sc as plsc

_E, _K, _H, _M, _SH = 64, 6, 128, 80, 160
_N = 2048
_T = 256
_CAP = 768
_TB = 256
_SUB = _CAP // _TB
_NBLK = _E * _SUB
_NS = _NBLK * _TB
_NW = 32
_TW = _N // _NW


def _router_kernel(x_ref, r_ref, wg_ref, tri_ref, wsg_ref, wsd_ref,
                   scores_ref, ybase_ref, slots_ref, wb0, wb1, wb2, wb3,
                   wb4, wb5, counts_ref, carry_ref):
    wtk_ref = (wb0, wb1, wb2, wb3, wb4, wb5)
    i = pl.program_id(0)
    x = x_ref[...]
    r = r_ref[...]

    logits = jnp.dot(r, wg_ref[...], preferred_element_type=jnp.float32)
    mx = jnp.max(logits, axis=1, keepdims=True)
    ex = jnp.exp(logits - mx)
    scores = ex / jnp.sum(ex, axis=1, keepdims=True)
    scores_ref[...] = scores

    iota = lax.broadcasted_iota(jnp.int32, scores.shape, 1)
    iota_f = iota.astype(jnp.float32)
    sbits = lax.bitcast_convert_type(scores, jnp.int32)
    key = lax.bitwise_or(lax.bitwise_and(sbits, ~jnp.int32(_E - 1)),
                         (_E - 1) - iota)
    picks = []
    sel = jnp.zeros(scores.shape, jnp.bool_)
    for _ in range(_K):
        m = jnp.max(key, axis=1, keepdims=True)
        pick = key == m
        picks.append(pick)
        sel = jnp.logical_or(sel, pick)
        key = jnp.where(pick, jnp.int32(-1), key)

    @pl.when(i == 0)
    def _init():
        carry_ref[...] = jnp.zeros_like(carry_ref)

    carry = carry_ref[0:1, :_E]
    sel_f = sel.astype(jnp.float32)
    rank_all = jnp.dot(tri_ref[...], sel_f,
                       preferred_element_type=jnp.float32) + carry
    new_counts = carry + jnp.sum(sel_f, axis=0, keepdims=True)
    carry_ref[0:1, :_E] = new_counts
    cpad = jnp.concatenate(
        [new_counts, jnp.zeros((1, _H - _E), jnp.float32)], axis=1)
    counts_ref[...] = cpad[None].astype(jnp.int32)

    lane7 = lax.broadcasted_iota(jnp.int32, (_T, _H), 1)
    slots_acc = jnp.zeros((_T, _H), jnp.float32)
    wks = []
    wsum = jnp.zeros((_T, 1), jnp.float32)
    for k in range(_K):
        p = picks[k].astype(jnp.float32)
        eid = jnp.sum(p * iota_f, axis=1, keepdims=True)
        rk = jnp.sum(p * rank_all, axis=1, keepdims=True)
        rk = jnp.minimum(rk, _CAP - 1.0)
        wk = jnp.sum(p * scores, axis=1, keepdims=True)
        pos = eid * _CAP + rk
        slots_acc = slots_acc + jnp.where(lane7 == k, pos, 0.0)
        wks.append(wk)
        wsum = wsum + wk
    slots_ref[...] = slots_acc.astype(jnp.int32)
    inv = 1.0 / (wsum + 1e-20)
    ones = jnp.ones((1, _H), jnp.float32)
    for k in range(_K):
        wtk_ref[k][...] = (wks[k] * inv) * ones

    xb = x.astype(jnp.bfloat16)
    sh = jnp.dot(xb, wsg_ref[...],
                 preferred_element_type=jnp.float32).astype(jnp.bfloat16)
    sg = sh[:, :_SH]
    su = sh[:, _SH:]
    sact = (sg + sg * jnp.tanh(sg)) * su
    ybase_ref[...] = x + jnp.dot(sact, wsd_ref[...],
                                 preferred_element_type=jnp.float32)


def _grouped_ffn_kernel(cnt_ref, xs_ref, w_ref, wgu_ref, wd_ref, out_ref):
    b = pl.program_id(0)
    e = b // _SUB
    sub = b % _SUB
    valid = jnp.minimum(cnt_ref[e], _CAP) - sub * _TB

    @pl.when(valid > 0)
    def _():
        riota = lax.broadcasted_iota(jnp.int32, (_TB, 1), 0)
        xm = jnp.where(riota < valid, xs_ref[...], 0.0).astype(jnp.bfloat16)
        h = jnp.dot(xm, wgu_ref[0],
                    preferred_element_type=jnp.float32).astype(jnp.bfloat16)
        h1 = h[:, :_M]
        h2 = h[:, _M:]
        act = (h1 + h1 * jnp.tanh(h1)) * h2
        o = jnp.dot(act, wd_ref[0], preferred_element_type=jnp.float32)
        out_ref[...] = o * w_ref[...]


def _sc_dispatch(slots, x, wbs):
    mesh = plsc.VectorSubcoreMesh(core_axis_name="c", subcore_axis_name="s")

    @functools.partial(
        pl.kernel, mesh=mesh,
        out_type=[jax.ShapeDtypeStruct((_NS, _H), jnp.float32),
                  jax.ShapeDtypeStruct((_NS, _H), jnp.float32)],
        scratch_types=[pltpu.VMEM((_TW, _H), jnp.int32),
                       pltpu.VMEM((_TW, _H), jnp.float32),
                       pltpu.VMEM((_TW, _H), jnp.float32),
                       pltpu.VMEM((_K, _TW), jnp.int32)],
        compiler_params=pltpu.CompilerParams(needs_layout_passes=False),
    )
    def dispatch(slots_hbm, x_hbm, w0, w1, w2, w3, w4, w5,
                 xs_hbm, ws_hbm, slots_v, x_v, w_v, pos_v):
        w_hbms = (w0, w1, w2, w3, w4, w5)
        wid = lax.axis_index("s") * 2 + lax.axis_index("c")
        base = wid * _TW
        pltpu.sync_copy(slots_hbm.at[pl.ds(base, _TW)], slots_v)
        pltpu.sync_copy(x_hbm.at[pl.ds(base, _TW)], x_v)
        nch = _TW // 16
        for c in range(nch):
            rows = lax.iota(jnp.int32, 16) + c * 16
            for k in range(_K):
                col = jnp.zeros((16,), jnp.int32) + k
                pos = plsc.load_gather(slots_v, [rows, col])
                pos_v[k, pl.ds(c * 16, 16)] = pos
        for k in range(_K):
            pltpu.sync_copy(x_v, xs_hbm.at[pos_v.at[k]])
            pltpu.sync_copy(w_hbms[k].at[pl.ds(base, _TW)], w_v)
            pltpu.sync_copy(w_v, ws_hbm.at[pos_v.at[k]])
        return None

    return dispatch(slots, x, *wbs)


def _sc_combine(slots, outs, ybase):
    mesh = plsc.VectorSubcoreMesh(core_axis_name="c", subcore_axis_name="s")
    nch = _TW // 16

    @functools.partial(
        pl.kernel, mesh=mesh,
        out_type=jax.ShapeDtypeStruct((_N, _H), jnp.float32),
        scratch_types=[pltpu.VMEM((_TW, _H), jnp.int32),
                       pltpu.VMEM((_K // 2, 128), jnp.int32),
                       pltpu.VMEM((_K // 2, 128, _H), jnp.float32),
                       pltpu.VMEM((_TW, _H), jnp.float32),
                       pltpu.VMEM((_TW, _H), jnp.float32)],
        compiler_params=pltpu.CompilerParams(needs_layout_passes=False),
    )
    def combine(slots_hbm, outs_hbm, ybase_hbm, y_hbm,
                slots_v, pos_rows, gbuf, ybase_v, yout_v):
        wid = lax.axis_index("s") * 2 + lax.axis_index("c")
        base = wid * _TW
        pltpu.sync_copy(slots_hbm.at[pl.ds(base, _TW)], slots_v)
        pltpu.sync_copy(ybase_hbm.at[pl.ds(base, _TW)], ybase_v)
        for k in range(_K):
            for c in range(nch):
                rows = lax.iota(jnp.int32, 16) + c * 16
                col = jnp.zeros((16,), jnp.int32) + k
                pos = plsc.load_gather(slots_v, [rows, col])
                off = (k % 2) * _TW + c * 16
                pos_rows[k // 2, pl.ds(off, 16)] = pos
        for j in range(_K // 2):
            pltpu.sync_copy(outs_hbm.at[pos_rows.at[j]], gbuf.at[j])

        def body(t, acc):
            for v in range(_H // 16):
                s = ybase_v[t, pl.ds(v * 16, 16)]
                for k in range(_K):
                    s = s + gbuf[k // 2, (k % 2) * _TW + t, pl.ds(v * 16, 16)]
                yout_v[t, pl.ds(v * 16, 16)] = s
            return acc

        lax.fori_loop(0, _TW, body, 0)
        pltpu.sync_copy(yout_v, y_hbm.at[pl.ds(base, _TW)])
        return None

    return combine(slots, outs, ybase)


def kernel(hidden_states, tgt_route, W_gate, Wg, Wu, Wd, Ws_g, Ws_u, Ws_d):
    B, S, H = hidden_states.shape
    x = hidden_states.reshape(_N, H)
    r = tgt_route.reshape(_N, H)

    wgT = W_gate.T
    tri = (jnp.arange(_T)[:, None] > jnp.arange(_T)[None, :]
           ).astype(jnp.float32)
    wsguT = jnp.concatenate([0.5 * Ws_g.T, Ws_u.T],
                            axis=1).astype(jnp.bfloat16)
    wsdT = Ws_d.T.astype(jnp.bfloat16)
    wgu = jnp.concatenate([0.5 * Wg.transpose(0, 2, 1),
                           Wu.transpose(0, 2, 1)],
                          axis=2).astype(jnp.bfloat16)
    wd = Wd.transpose(0, 2, 1).astype(jnp.bfloat16)

    grid = (_N // _T,)
    tok = lambda i: (i, 0)
    full = lambda i: (0, 0)
    router_out = pl.pallas_call(
        _router_kernel,
        grid=grid,
        in_specs=[
            pl.BlockSpec((_T, H), tok),
            pl.BlockSpec((_T, H), tok),
            pl.BlockSpec((H, _E), full),
            pl.BlockSpec((_T, _T), full),
            pl.BlockSpec((H, 2 * _SH), full),
            pl.BlockSpec((_SH, H), full),
        ],
        out_specs=[
            pl.BlockSpec((_T, _E), tok),
            pl.BlockSpec((_T, H), tok),
            pl.BlockSpec((_T, H), tok),
        ] + [pl.BlockSpec((_T, H), tok)] * _K + [
            pl.BlockSpec((1, 1, H), lambda i: (i, 0, 0)),
        ],
        out_shape=[
            jax.ShapeDtypeStruct((_N, _E), jnp.float32),
            jax.ShapeDtypeStruct((_N, H), jnp.float32),
            jax.ShapeDtypeStruct((_N, H), jnp.int32),
        ] + [jax.ShapeDtypeStruct((_N, H), jnp.float32)] * _K + [
            jax.ShapeDtypeStruct((_N // _T, 1, H), jnp.int32),
        ],
        scratch_shapes=[pltpu.VMEM((8, 128), jnp.float32)],
        compiler_params=pltpu.CompilerParams(
            dimension_semantics=("arbitrary",)),
    )(x, r, wgT, tri, wsguT, wsdT)
    scores, ybase, slots = router_out[0], router_out[1], router_out[2]
    wbs = router_out[3:3 + _K]
    counts3 = router_out[3 + _K]

    counts = counts3[_N // _T - 1, 0, :_E]

    xs, ws = _sc_dispatch(slots, x, wbs)

    def _xs_map(b, cnt):
        e = b // _SUB
        live = jnp.minimum(cnt[e], _CAP) > (b % _SUB) * _TB
        return (jnp.where(live, b, 0), 0)

    outs = pl.pallas_call(
        _grouped_ffn_kernel,
        grid_spec=pltpu.PrefetchScalarGridSpec(
            num_scalar_prefetch=1,
            grid=(_NBLK,),
            in_specs=[
                pl.BlockSpec((_TB, H), _xs_map),
                pl.BlockSpec((_TB, H), _xs_map),
                pl.BlockSpec((1, H, 2 * _M), lambda b, c: (b // _SUB, 0, 0)),
                pl.BlockSpec((1, _M, H), lambda b, c: (b // _SUB, 0, 0)),
            ],
            out_specs=pl.BlockSpec((_TB, H), lambda b, c: (b, 0)),
        ),
        out_shape=jax.ShapeDtypeStruct((_NS, H), jnp.float32),
        compiler_params=pltpu.CompilerParams(
            dimension_semantics=("arbitrary",)),
    )(counts, xs, ws, wgu, wd)

    y = _sc_combine(slots, outs, ybase)
    return y.reshape(B, S, H), scores


# speedup vs baseline: 1.6607x; 1.6607x over previous
import functools

import jax
import jax.numpy as jnp
from jax import lax
from jax.experimental import pallas as pl
from jax.experimental.pallas import tpu as pltpu
from jax.experimental.pallas import tpu_sc as plsc

_E, _K, _H, _M, _SH = 64, 6, 128, 80, 160
_N = 2048
_T = 256
_CAP = 512
_TB = 512
_SUB = _CAP // _TB
_NBLK = _E * _SUB
_NS = _NBLK * _TB
_NW = 32
_TW = _N // _NW


def _router_kernel(x_ref, r_ref, wg_ref, tri_ref, wsg_ref, wsd_ref,
                   scores_ref, ybase_ref, slots_ref, wb0, wb1, wb2, wb3,
                   wb4, wb5, counts_ref, carry_ref):
    wtk_ref = (wb0, wb1, wb2, wb3, wb4, wb5)
    i = pl.program_id(0)
    x = x_ref[...]
    r = r_ref[...]

    logits = jnp.dot(r, wg_ref[...], preferred_element_type=jnp.float32)
    mx = jnp.max(logits, axis=1, keepdims=True)
    ex = jnp.exp(logits - mx)
    scores = ex / jnp.sum(ex, axis=1, keepdims=True)
    scores_ref[...] = scores

    iota = lax.broadcasted_iota(jnp.int32, scores.shape, 1)
    iota_f = iota.astype(jnp.float32)
    sbits = lax.bitcast_convert_type(scores, jnp.int32)
    key = lax.bitwise_or(lax.bitwise_and(sbits, ~jnp.int32(_E - 1)),
                         (_E - 1) - iota)
    picks = []
    sel = jnp.zeros(scores.shape, jnp.bool_)
    for _ in range(_K):
        m = jnp.max(key, axis=1, keepdims=True)
        pick = key == m
        picks.append(pick)
        sel = jnp.logical_or(sel, pick)
        key = jnp.where(pick, jnp.int32(-1), key)

    @pl.when(i == 0)
    def _init():
        carry_ref[...] = jnp.zeros_like(carry_ref)

    carry = carry_ref[0:1, :_E]
    sel_f = sel.astype(jnp.float32)
    rank_all = jnp.dot(tri_ref[...], sel_f,
                       preferred_element_type=jnp.float32) + carry
    new_counts = carry + jnp.sum(sel_f, axis=0, keepdims=True)
    carry_ref[0:1, :_E] = new_counts
    cpad = jnp.concatenate(
        [new_counts, jnp.zeros((1, _H - _E), jnp.float32)], axis=1)
    counts_ref[...] = cpad[None].astype(jnp.int32)

    lane7 = lax.broadcasted_iota(jnp.int32, (_T, _H), 1)
    slots_acc = jnp.zeros((_T, _H), jnp.float32)
    wks = []
    wsum = jnp.zeros((_T, 1), jnp.float32)
    for k in range(_K):
        p = picks[k].astype(jnp.float32)
        eid = jnp.sum(p * iota_f, axis=1, keepdims=True)
        rk = jnp.sum(p * rank_all, axis=1, keepdims=True)
        rk = jnp.minimum(rk, _CAP - 1.0)
        wk = jnp.sum(p * scores, axis=1, keepdims=True)
        pos = eid * _CAP + rk
        slots_acc = slots_acc + jnp.where(lane7 == k, pos, 0.0)
        wks.append(wk)
        wsum = wsum + wk
    slots_ref[...] = slots_acc.astype(jnp.int32)
    inv = 1.0 / (wsum + 1e-20)
    ones = jnp.ones((1, _H), jnp.float32)
    for k in range(_K):
        wtk_ref[k][...] = (wks[k] * inv) * ones

    xb = x.astype(jnp.bfloat16)
    sh = jnp.dot(xb, wsg_ref[...],
                 preferred_element_type=jnp.float32).astype(jnp.bfloat16)
    sg = sh[:, :_SH]
    su = sh[:, _SH:]
    sact = (sg + sg * jnp.tanh(sg)) * su
    ybase_ref[...] = x + jnp.dot(sact, wsd_ref[...],
                                 preferred_element_type=jnp.float32)


def _grouped_ffn_kernel(cnt_ref, xs_ref, w_ref, wgu_ref, wd_ref, out_ref):
    b = pl.program_id(0)
    e = b // _SUB
    sub = b % _SUB
    valid = jnp.minimum(cnt_ref[e], _CAP) - sub * _TB

    @pl.when(valid > 0)
    def _():
        riota = lax.broadcasted_iota(jnp.int32, (_TB, 1), 0)
        xm = jnp.where(riota < valid, xs_ref[...], 0.0).astype(jnp.bfloat16)
        h = jnp.dot(xm, wgu_ref[0],
                    preferred_element_type=jnp.float32).astype(jnp.bfloat16)
        h1 = h[:, :_M]
        h2 = h[:, _M:]
        act = (h1 + h1 * jnp.tanh(h1)) * h2
        o = jnp.dot(act, wd_ref[0], preferred_element_type=jnp.float32)
        out_ref[...] = o * w_ref[...]


def _sc_dispatch(slots, x, wbs):
    mesh = plsc.VectorSubcoreMesh(core_axis_name="c", subcore_axis_name="s")

    @functools.partial(
        pl.kernel, mesh=mesh,
        out_type=[jax.ShapeDtypeStruct((_NS, _H), jnp.float32),
                  jax.ShapeDtypeStruct((_NS, _H), jnp.float32)],
        scratch_types=[pltpu.VMEM((_TW, _H), jnp.int32),
                       pltpu.VMEM((_TW, _H), jnp.float32),
                       pltpu.VMEM((_K, _TW, _H), jnp.float32),
                       pltpu.VMEM((_K, _TW), jnp.int32),
                       pltpu.SemaphoreType.DMA,
                       pltpu.SemaphoreType.DMA],
        compiler_params=pltpu.CompilerParams(needs_layout_passes=False),
    )
    def dispatch(slots_hbm, x_hbm, w0, w1, w2, w3, w4, w5,
                 xs_hbm, ws_hbm, slots_v, x_v, w_v, pos_v,
                 sem_in, sem_out):
        w_hbms = (w0, w1, w2, w3, w4, w5)
        wid = lax.axis_index("s") * 2 + lax.axis_index("c")
        base = wid * _TW
        win = []
        for k in range(_K):
            win.append(pltpu.async_copy(
                w_hbms[k].at[pl.ds(base, _TW)], w_v.at[k], sem_in))
        pltpu.sync_copy(slots_hbm.at[pl.ds(base, _TW)], slots_v)
        pltpu.sync_copy(x_hbm.at[pl.ds(base, _TW)], x_v)
        nch = _TW // 16
        for c in range(nch):
            rows = lax.iota(jnp.int32, 16) + c * 16
            for k in range(_K):
                col = jnp.zeros((16,), jnp.int32) + k
                pos = plsc.load_gather(slots_v, [rows, col])
                pos_v[k, pl.ds(c * 16, 16)] = pos
        wout = []
        for k in range(_K):
            wout.append(pltpu.async_copy(x_v, xs_hbm.at[pos_v.at[k]],
                                         sem_out))
        for h in win:
            h.wait()
        for k in range(_K):
            wout.append(pltpu.async_copy(w_v.at[k], ws_hbm.at[pos_v.at[k]],
                                         sem_out))
        for h in wout:
            h.wait()
        return None

    return dispatch(slots, x, *wbs)


def _sc_combine(slots, outs, ybase):
    mesh = plsc.VectorSubcoreMesh(core_axis_name="c", subcore_axis_name="s")
    nch = _TW // 16

    @functools.partial(
        pl.kernel, mesh=mesh,
        out_type=jax.ShapeDtypeStruct((_N, _H), jnp.float32),
        scratch_types=[pltpu.VMEM((_TW, _H), jnp.int32),
                       pltpu.VMEM((_K // 2, 128), jnp.int32),
                       pltpu.VMEM((_K // 2, 128, _H), jnp.float32),
                       pltpu.VMEM((_TW, _H), jnp.float32),
                       pltpu.VMEM((_TW, _H), jnp.float32),
                       pltpu.SemaphoreType.DMA,
                       pltpu.SemaphoreType.DMA],
        compiler_params=pltpu.CompilerParams(needs_layout_passes=False),
    )
    def combine(slots_hbm, outs_hbm, ybase_hbm, y_hbm,
                slots_v, pos_rows, gbuf, ybase_v, yout_v, sem_b, sem_g):
        wid = lax.axis_index("s") * 2 + lax.axis_index("c")
        base = wid * _TW
        hb = pltpu.async_copy(ybase_hbm.at[pl.ds(base, _TW)], ybase_v, sem_b)
        pltpu.sync_copy(slots_hbm.at[pl.ds(base, _TW)], slots_v)
        for k in range(_K):
            for c in range(nch):
                rows = lax.iota(jnp.int32, 16) + c * 16
                col = jnp.zeros((16,), jnp.int32) + k
                pos = plsc.load_gather(slots_v, [rows, col])
                off = (k % 2) * _TW + c * 16
                pos_rows[k // 2, pl.ds(off, 16)] = pos
        hg = [pltpu.async_copy(outs_hbm.at[pos_rows.at[j]], gbuf.at[j], sem_g)
              for j in range(_K // 2)]
        hb.wait()
        for h in hg:
            h.wait()

        def body(t, acc):
            for v in range(_H // 16):
                s = ybase_v[t, pl.ds(v * 16, 16)]
                for k in range(_K):
                    s = s + gbuf[k // 2, (k % 2) * _TW + t, pl.ds(v * 16, 16)]
                yout_v[t, pl.ds(v * 16, 16)] = s
            return acc

        lax.fori_loop(0, _TW, body, 0)
        pltpu.sync_copy(yout_v, y_hbm.at[pl.ds(base, _TW)])
        return None

    return combine(slots, outs, ybase)


def kernel(hidden_states, tgt_route, W_gate, Wg, Wu, Wd, Ws_g, Ws_u, Ws_d):
    B, S, H = hidden_states.shape
    x = hidden_states.reshape(_N, H)
    r = tgt_route.reshape(_N, H)

    wgT = W_gate.T
    tri = (jnp.arange(_T)[:, None] > jnp.arange(_T)[None, :]
           ).astype(jnp.float32)
    wsguT = jnp.concatenate([0.5 * Ws_g.T, Ws_u.T],
                            axis=1).astype(jnp.bfloat16)
    wsdT = Ws_d.T.astype(jnp.bfloat16)
    wgu = jnp.concatenate([0.5 * Wg.transpose(0, 2, 1),
                           Wu.transpose(0, 2, 1)],
                          axis=2).astype(jnp.bfloat16)
    wd = Wd.transpose(0, 2, 1).astype(jnp.bfloat16)

    grid = (_N // _T,)
    tok = lambda i: (i, 0)
    full = lambda i: (0, 0)
    router_out = pl.pallas_call(
        _router_kernel,
        grid=grid,
        in_specs=[
            pl.BlockSpec((_T, H), tok),
            pl.BlockSpec((_T, H), tok),
            pl.BlockSpec((H, _E), full),
            pl.BlockSpec((_T, _T), full),
            pl.BlockSpec((H, 2 * _SH), full),
            pl.BlockSpec((_SH, H), full),
        ],
        out_specs=[
            pl.BlockSpec((_T, _E), tok),
            pl.BlockSpec((_T, H), tok),
            pl.BlockSpec((_T, H), tok),
        ] + [pl.BlockSpec((_T, H), tok)] * _K + [
            pl.BlockSpec((1, 1, H), lambda i: (i, 0, 0)),
        ],
        out_shape=[
            jax.ShapeDtypeStruct((_N, _E), jnp.float32),
            jax.ShapeDtypeStruct((_N, H), jnp.float32),
            jax.ShapeDtypeStruct((_N, H), jnp.int32),
        ] + [jax.ShapeDtypeStruct((_N, H), jnp.float32)] * _K + [
            jax.ShapeDtypeStruct((_N // _T, 1, H), jnp.int32),
        ],
        scratch_shapes=[pltpu.VMEM((8, 128), jnp.float32)],
        compiler_params=pltpu.CompilerParams(
            dimension_semantics=("arbitrary",)),
    )(x, r, wgT, tri, wsguT, wsdT)
    scores, ybase, slots = router_out[0], router_out[1], router_out[2]
    wbs = router_out[3:3 + _K]
    counts3 = router_out[3 + _K]

    counts = counts3[_N // _T - 1, 0, :_E]

    xs, ws = _sc_dispatch(slots, x, wbs)

    def _xs_map(b, cnt):
        e = b // _SUB
        live = jnp.minimum(cnt[e], _CAP) > (b % _SUB) * _TB
        return (jnp.where(live, b, 0), 0)

    outs = pl.pallas_call(
        _grouped_ffn_kernel,
        grid_spec=pltpu.PrefetchScalarGridSpec(
            num_scalar_prefetch=1,
            grid=(_NBLK,),
            in_specs=[
                pl.BlockSpec((_TB, H), _xs_map),
                pl.BlockSpec((_TB, H), _xs_map),
                pl.BlockSpec((1, H, 2 * _M), lambda b, c: (b // _SUB, 0, 0)),
                pl.BlockSpec((1, _M, H), lambda b, c: (b // _SUB, 0, 0)),
            ],
            out_specs=pl.BlockSpec((_TB, H), lambda b, c: (b, 0)),
        ),
        out_shape=jax.ShapeDtypeStruct((_NS, H), jnp.float32),
        compiler_params=pltpu.CompilerParams(
            dimension_semantics=("arbitrary",)),
    )(counts, xs, ws, wgu, wd)

    y = _sc_combine(slots, outs, ybase)
    return y.reshape(B, S, H), scores


# speedup vs baseline: 4.4830x; 2.6995x over previous
import jax
import jax.numpy as jnp
from jax.experimental import pallas as pl
from jax.experimental.pallas import tpu as pltpu

_E, _K, _H, _M, _SH = 64, 6, 128, 80, 160
_T = 512


def _moe_block_kernel(x_ref, r_ref, wg_ref, wgt_ref, wdt_ref,
                      rmap_ref, wsg_ref, wsd_ref, y_ref, scores_ref):
    x = x_ref[...]
    r = r_ref[...]

    logits = jnp.dot(r, wg_ref[...], preferred_element_type=jnp.float32)
    mx = jnp.max(logits, axis=1, keepdims=True)
    ex = jnp.exp(logits - mx)
    scores = ex / jnp.sum(ex, axis=1, keepdims=True)
    scores_ref[...] = scores

    iota = jax.lax.broadcasted_iota(jnp.int32, scores.shape, 1)
    sbits = jax.lax.bitcast_convert_type(scores, jnp.int32)
    key = jax.lax.bitwise_or(jax.lax.bitwise_and(sbits, ~jnp.int32(_E - 1)),
                             (_E - 1) - iota)
    sel = jnp.zeros(scores.shape, jnp.bool_)
    for _ in range(_K):
        m = jnp.max(key, axis=1, keepdims=True)
        pick = key == m
        sel = jnp.logical_or(sel, pick)
        key = jnp.where(pick, jnp.int32(-1), key)
    wts = jnp.where(sel, scores, 0.0)
    wts = wts / (jnp.sum(wts, axis=1, keepdims=True) + 1e-20)

    xb = x.astype(jnp.bfloat16)
    h = jnp.dot(xb, wgt_ref[...],
                preferred_element_type=jnp.float32).astype(jnp.bfloat16)
    h1 = h[:, :_E * _M]
    h2 = h[:, _E * _M:]
    half = jnp.bfloat16(0.5)
    hh = half * h1
    act = (hh + hh * jnp.tanh(hh)) * h2
    wwide = jnp.dot(wts.astype(jnp.bfloat16), rmap_ref[...],
                    preferred_element_type=jnp.float32).astype(jnp.bfloat16)
    scaled = act * wwide
    y = jnp.dot(scaled, wdt_ref[...], preferred_element_type=jnp.float32)

    sh = jnp.dot(xb, wsg_ref[...],
                 preferred_element_type=jnp.float32).astype(jnp.bfloat16)
    sg = sh[:, :_SH]
    su = sh[:, _SH:]
    shh = half * sg
    sact = (shh + shh * jnp.tanh(shh)) * su
    y = y + jnp.dot(sact, wsd_ref[...], preferred_element_type=jnp.float32)

    y_ref[...] = y + x


def kernel(hidden_states, tgt_route, W_gate, Wg, Wu, Wd, Ws_g, Ws_u, Ws_d):
    B, S, H = hidden_states.shape
    N = B * S
    x = hidden_states.reshape(N, H)
    r = tgt_route.reshape(N, H)

    wgT = W_gate.T
    wgtT = Wg.transpose(2, 0, 1).reshape(H, _E * _M).astype(jnp.bfloat16)
    wutT = Wu.transpose(2, 0, 1).reshape(H, _E * _M).astype(jnp.bfloat16)
    wguT = jnp.concatenate([wgtT, wutT], axis=1)
    wdtT = Wd.transpose(0, 2, 1).reshape(_E * _M, H).astype(jnp.bfloat16)
    rmap = (jnp.arange(_E)[:, None] == (jnp.arange(_E * _M)[None, :] // _M)
            ).astype(jnp.bfloat16)
    wsguT = jnp.concatenate([Ws_g.T, Ws_u.T], axis=1).astype(jnp.bfloat16)
    wsdT = Ws_d.T.astype(jnp.bfloat16)

    grid = (N // _T,)
    tok = lambda i: (i, 0)
    full = lambda i: (0, 0)
    y, scores = pl.pallas_call(
        _moe_block_kernel,
        grid=grid,
        in_specs=[
            pl.BlockSpec((_T, H), tok),
            pl.BlockSpec((_T, H), tok),
            pl.BlockSpec((H, _E), full),
            pl.BlockSpec((H, 2 * _E * _M), full),
            pl.BlockSpec((_E * _M, H), full),
            pl.BlockSpec((_E, _E * _M), full),
            pl.BlockSpec((H, 2 * _SH), full),
            pl.BlockSpec((_SH, H), full),
        ],
        out_specs=[
            pl.BlockSpec((_T, H), tok),
            pl.BlockSpec((_T, _E), tok),
        ],
        out_shape=[
            jax.ShapeDtypeStruct((N, H), jnp.float32),
            jax.ShapeDtypeStruct((N, _E), jnp.float32),
        ],
        compiler_params=pltpu.CompilerParams(
            dimension_semantics=("parallel",)),
    )(x, r, wgT, wguT, wdtT, rmap, wsguT, wsdT)
    return y.reshape(B, S, H), scores


# speedup vs baseline: 4.5505x; 1.0151x over previous
import jax
import jax.numpy as jnp
from jax.experimental import pallas as pl
from jax.experimental.pallas import tpu as pltpu

_E, _K, _H, _M, _SH = 64, 6, 128, 80, 160
_T = 1024


def _moe_block_kernel(x_ref, r_ref, wg_ref, wgt_ref, wdt_ref,
                      rmap_ref, wsg_ref, wsd_ref, y_ref, scores_ref):
    x = x_ref[...]
    r = r_ref[...]

    logits = jnp.dot(r, wg_ref[...], preferred_element_type=jnp.float32)
    mx = jnp.max(logits, axis=1, keepdims=True)
    ex = jnp.exp(logits - mx)
    scores = ex / jnp.sum(ex, axis=1, keepdims=True)
    scores_ref[...] = scores

    iota = jax.lax.broadcasted_iota(jnp.int32, scores.shape, 1)
    sbits = jax.lax.bitcast_convert_type(scores, jnp.int32)
    key = jax.lax.bitwise_or(jax.lax.bitwise_and(sbits, ~jnp.int32(_E - 1)),
                             (_E - 1) - iota)
    sel = jnp.zeros(scores.shape, jnp.bool_)
    for _ in range(_K):
        m = jnp.max(key, axis=1, keepdims=True)
        pick = key == m
        sel = jnp.logical_or(sel, pick)
        key = jnp.where(pick, jnp.int32(-1), key)
    wts = jnp.where(sel, scores, 0.0)
    wts = wts / (jnp.sum(wts, axis=1, keepdims=True) + 1e-20)

    xb = x.astype(jnp.bfloat16)
    h = jnp.dot(xb, wgt_ref[...],
                preferred_element_type=jnp.float32).astype(jnp.bfloat16)
    h1 = h[:, :_E * _M]
    h2 = h[:, _E * _M:]
    half = jnp.bfloat16(0.5)
    hh = half * h1
    act = (hh + hh * jnp.tanh(hh)) * h2
    wwide = jnp.dot(wts.astype(jnp.bfloat16), rmap_ref[...],
                    preferred_element_type=jnp.float32).astype(jnp.bfloat16)
    scaled = act * wwide
    y = jnp.dot(scaled, wdt_ref[...], preferred_element_type=jnp.float32)

    sh = jnp.dot(xb, wsg_ref[...],
                 preferred_element_type=jnp.float32).astype(jnp.bfloat16)
    sg = sh[:, :_SH]
    su = sh[:, _SH:]
    shh = half * sg
    sact = (shh + shh * jnp.tanh(shh)) * su
    y = y + jnp.dot(sact, wsd_ref[...], preferred_element_type=jnp.float32)

    y_ref[...] = y + x


def kernel(hidden_states, tgt_route, W_gate, Wg, Wu, Wd, Ws_g, Ws_u, Ws_d):
    B, S, H = hidden_states.shape
    N = B * S
    x = hidden_states.reshape(N, H)
    r = tgt_route.reshape(N, H)

    wgT = W_gate.T
    wgtT = Wg.transpose(2, 0, 1).reshape(H, _E * _M).astype(jnp.bfloat16)
    wutT = Wu.transpose(2, 0, 1).reshape(H, _E * _M).astype(jnp.bfloat16)
    wguT = jnp.concatenate([wgtT, wutT], axis=1)
    wdtT = Wd.transpose(0, 2, 1).reshape(_E * _M, H).astype(jnp.bfloat16)
    rmap = (jnp.arange(_E)[:, None] == (jnp.arange(_E * _M)[None, :] // _M)
            ).astype(jnp.bfloat16)
    wsguT = jnp.concatenate([Ws_g.T, Ws_u.T], axis=1).astype(jnp.bfloat16)
    wsdT = Ws_d.T.astype(jnp.bfloat16)

    grid = (N // _T,)
    tok = lambda i: (i, 0)
    full = lambda i: (0, 0)
    y, scores = pl.pallas_call(
        _moe_block_kernel,
        grid=grid,
        in_specs=[
            pl.BlockSpec((_T, H), tok),
            pl.BlockSpec((_T, H), tok),
            pl.BlockSpec((H, _E), full),
            pl.BlockSpec((H, 2 * _E * _M), full),
            pl.BlockSpec((_E * _M, H), full),
            pl.BlockSpec((_E, _E * _M), full),
            pl.BlockSpec((H, 2 * _SH), full),
            pl.BlockSpec((_SH, H), full),
        ],
        out_specs=[
            pl.BlockSpec((_T, H), tok),
            pl.BlockSpec((_T, _E), tok),
        ],
        out_shape=[
            jax.ShapeDtypeStruct((N, H), jnp.float32),
            jax.ShapeDtypeStruct((N, _E), jnp.float32),
        ],
        compiler_params=pltpu.CompilerParams(
            dimension_semantics=("parallel",)),
    )(x, r, wgT, wguT, wdtT, rmap, wsguT, wsdT)
    return y.reshape(B, S, H), scores


# speedup vs baseline: 4.5512x; 1.0002x over previous
import jax
import jax.numpy as jnp
from jax.experimental import pallas as pl
from jax.experimental.pallas import tpu as pltpu

_E, _K, _H, _M, _SH = 64, 6, 128, 80, 160
_T = 1024


def _moe_block_kernel(x_ref, r_ref, wg_ref, wgt_ref, wdt_ref,
                      rmap_ref, wsg_ref, wsd_ref, y_ref, scores_ref):
    x = x_ref[...]
    r = r_ref[...]

    logits = jnp.dot(r, wg_ref[...], preferred_element_type=jnp.float32)
    mx = jnp.max(logits, axis=1, keepdims=True)
    ex = jnp.exp(logits - mx)
    scores = ex / jnp.sum(ex, axis=1, keepdims=True)
    scores_ref[...] = scores

    iota = jax.lax.broadcasted_iota(jnp.int32, scores.shape, 1)
    sbits = jax.lax.bitcast_convert_type(scores, jnp.int32)
    key = jax.lax.bitwise_or(jax.lax.bitwise_and(sbits, ~jnp.int32(_E - 1)),
                             (_E - 1) - iota)
    sel = jnp.zeros(scores.shape, jnp.bool_)
    for _ in range(_K):
        m = jnp.max(key, axis=1, keepdims=True)
        pick = key == m
        sel = jnp.logical_or(sel, pick)
        key = jnp.where(pick, jnp.int32(-1), key)
    wts = jnp.where(sel, scores, 0.0)
    wts = wts / (jnp.sum(wts, axis=1, keepdims=True) + 1e-20)

    xb = x.astype(jnp.bfloat16)
    h = jnp.dot(xb, wgt_ref[...],
                preferred_element_type=jnp.float32).astype(jnp.bfloat16)
    h1 = h[:, :_E * _M]
    h2 = h[:, _E * _M:]
    act = (h1 + h1 * jnp.tanh(h1)) * h2
    wwide = jnp.dot(wts.astype(jnp.bfloat16), rmap_ref[...],
                    preferred_element_type=jnp.float32).astype(jnp.bfloat16)
    scaled = act * wwide
    y = jnp.dot(scaled, wdt_ref[...], preferred_element_type=jnp.float32)

    sh = jnp.dot(xb, wsg_ref[...],
                 preferred_element_type=jnp.float32).astype(jnp.bfloat16)
    sg = sh[:, :_SH]
    su = sh[:, _SH:]
    sact = (sg + sg * jnp.tanh(sg)) * su
    y = y + jnp.dot(sact, wsd_ref[...], preferred_element_type=jnp.float32)

    y_ref[...] = y + x


def kernel(hidden_states, tgt_route, W_gate, Wg, Wu, Wd, Ws_g, Ws_u, Ws_d):
    B, S, H = hidden_states.shape
    N = B * S
    x = hidden_states.reshape(N, H)
    r = tgt_route.reshape(N, H)

    wgT = W_gate.T
    wgtT = (0.5 * Wg.transpose(2, 0, 1).reshape(H, _E * _M)
            ).astype(jnp.bfloat16)
    wutT = Wu.transpose(2, 0, 1).reshape(H, _E * _M).astype(jnp.bfloat16)
    wguT = jnp.concatenate([wgtT, wutT], axis=1)
    wdtT = Wd.transpose(0, 2, 1).reshape(_E * _M, H).astype(jnp.bfloat16)
    rmap = (jnp.arange(_E)[:, None] == (jnp.arange(_E * _M)[None, :] // _M)
            ).astype(jnp.bfloat16)
    wsguT = jnp.concatenate([0.5 * Ws_g.T, Ws_u.T],
                            axis=1).astype(jnp.bfloat16)
    wsdT = Ws_d.T.astype(jnp.bfloat16)

    grid = (N // _T,)
    tok = lambda i: (i, 0)
    full = lambda i: (0, 0)
    y, scores = pl.pallas_call(
        _moe_block_kernel,
        grid=grid,
        in_specs=[
            pl.BlockSpec((_T, H), tok),
            pl.BlockSpec((_T, H), tok),
            pl.BlockSpec((H, _E), full),
            pl.BlockSpec((H, 2 * _E * _M), full),
            pl.BlockSpec((_E * _M, H), full),
            pl.BlockSpec((_E, _E * _M), full),
            pl.BlockSpec((H, 2 * _SH), full),
            pl.BlockSpec((_SH, H), full),
        ],
        out_specs=[
            pl.BlockSpec((_T, H), tok),
            pl.BlockSpec((_T, _E), tok),
        ],
        out_shape=[
            jax.ShapeDtypeStruct((N, H), jnp.float32),
            jax.ShapeDtypeStruct((N, _E), jnp.float32),
        ],
        compiler_params=pltpu.CompilerParams(
            dimension_semantics=("parallel",)),
    )(x, r, wgT, wguT, wdtT, rmap, wsguT, wsdT)
    return y.reshape(B, S, H), scores
